# baseline (speedup 1.0000x reference)
import functools

import jax
import jax.numpy as jnp
from jax import lax
from jax.experimental import pallas as pl
from jax.experimental.pallas import tpu as pltpu
from jax.experimental.pallas import tpu_sc as plsc

N = 10000
E = 320000
D = 128

NC = 2
NS = 16
NW = NC * NS
EPW = E // NW
CHUNK = 128
G = 8
NCHUNK = 80
NGROUP = NCHUNK // G
EPW_PAD = NCHUNK * CHUNK
NROW_ACC = 10240
ROWS_PER_TILE = NROW_ACC // NS
OUT_CHUNK = 128


def _sc_segment_sum(h, src3d, dst3d, zeros):
    mesh = plsc.VectorSubcoreMesh(
        core_axis_name="c", subcore_axis_name="s", num_cores=NC,
        num_subcores=NS)

    @functools.partial(
        pl.kernel,
        out_type=jax.ShapeDtypeStruct((NC, NROW_ACC, D), jnp.float32),
        mesh=mesh,
        scratch_types=[
            pltpu.VMEM((2 * G, CHUNK), jnp.int32),
            pltpu.VMEM((2 * G, CHUNK), jnp.int32),
            pltpu.VMEM((CHUNK, D), jnp.float32),
            pltpu.VMEM((CHUNK, D), jnp.float32),
            pltpu.VMEM_SHARED((NROW_ACC, D), jnp.float32),
            pltpu.SemaphoreType.DMA,
            pltpu.SemaphoreType.DMA,
            pltpu.SemaphoreType.DMA,
            pltpu.SemaphoreType.DMA,
            pltpu.SemaphoreType.DMA,
        ],
    )
    def seg_sum(h_hbm, src_hbm, dst_hbm, zeros_hbm, out_hbm,
                src_v, dst_v, rows0_v, rows1_v, agg_sh,
                isem, gsem0, gsem1, ssem0, ssem1):
        c = lax.axis_index("c")
        s = lax.axis_index("s")
        w = c * NS + s
        base = s * ROWS_PER_TILE
        rows = (rows0_v, rows1_v)
        gsem = (gsem0, gsem1)
        ssem = (ssem0, ssem1)
        pltpu.sync_copy(zeros_hbm.at[pl.ds(base, ROWS_PER_TILE)],
                        agg_sh.at[pl.ds(base, ROWS_PER_TILE)])
        pltpu.async_copy(src_hbm.at[w, pl.ds(0, G)],
                         src_v.at[pl.ds(0, G)], isem)
        pltpu.async_copy(dst_hbm.at[w, pl.ds(0, G)],
                         dst_v.at[pl.ds(0, G)], isem)
        plsc.subcore_barrier()

        def group_body(g, carry):
            gbase = lax.rem(g, 2) * G
            pltpu.make_async_copy(
                src_hbm.at[w, pl.ds(0, G)], src_v.at[pl.ds(0, G)],
                isem).wait()
            pltpu.make_async_copy(
                dst_hbm.at[w, pl.ds(0, G)], dst_v.at[pl.ds(0, G)],
                isem).wait()

            @pl.when(g + 1 < NGROUP)
            def _():
                off = (g + 1) * G
                nbase = lax.rem(g + 1, 2) * G
                pltpu.async_copy(src_hbm.at[w, pl.ds(off, G)],
                                 src_v.at[pl.ds(nbase, G)], isem)
                pltpu.async_copy(dst_hbm.at[w, pl.ds(off, G)],
                                 dst_v.at[pl.ds(nbase, G)], isem)

            gd = [None] * G
            sd = [None] * G
            gd[0] = pltpu.async_copy(
                h_hbm.at[src_v.at[gbase]], rows[0], gsem[0])
            for b in range(G):
                p = b % 2
                gd[b].wait()
                if b + 1 < G:
                    if b >= 1:
                        sd[b - 1].wait()
                    gd[b + 1] = pltpu.async_copy(
                        h_hbm.at[src_v.at[gbase + b + 1]], rows[1 - p],
                        gsem[1 - p])
                sd[b] = pltpu.async_copy(
                    rows[p], agg_sh.at[dst_v.at[gbase + b]], ssem[p],
                    add=True)
            sd[G - 2].wait()
            sd[G - 1].wait()
            return carry

        lax.fori_loop(0, NGROUP, group_body, 0)
        plsc.subcore_barrier()
        for k in range(ROWS_PER_TILE // OUT_CHUNK):
            off = base + k * OUT_CHUNK
            pltpu.sync_copy(agg_sh.at[pl.ds(off, OUT_CHUNK)], rows0_v)
            pltpu.sync_copy(rows0_v, out_hbm.at[c, pl.ds(off, OUT_CHUNK)])

    return seg_sum(h, src3d, dst3d, zeros)


_ROW_BLK = 1000


def _mlp_body(h_ref, p0_ref, p1_ref, Wa_ref, ba_ref, Wb_ref, bb_ref, o_ref):
    z = h_ref[...] + (p0_ref[...] + p1_ref[...])
    z = jnp.dot(z, Wa_ref[...], preferred_element_type=jnp.float32) + ba_ref[...]
    z = jnp.maximum(z, 0.0)
    z = jnp.dot(z, Wb_ref[...], preferred_element_type=jnp.float32) + bb_ref[...]
    o_ref[...] = jnp.maximum(z, 0.0)


def _tc_mlp(h, p0, p1, Wa, ba, Wb, bb):
    row = pl.BlockSpec((_ROW_BLK, D), lambda i: (i, 0))
    wsp = pl.BlockSpec((D, D), lambda i: (0, 0))
    bsp = pl.BlockSpec((1, D), lambda i: (0, 0))
    return pl.pallas_call(
        _mlp_body,
        grid=(N // _ROW_BLK,),
        in_specs=[row, row, row, wsp, bsp, wsp, bsp],
        out_specs=row,
        out_shape=jax.ShapeDtypeStruct((N, D), jnp.float32),
    )(h, p0, p1, Wa, ba.reshape(1, D), Wb, bb.reshape(1, D))


def _head_body(h_ref, p0_ref, p1_ref, x1_ref, x2_ref, Wa_ref, ba_ref,
               Wb_ref, bb_ref, Wm_ref, bm_ref, o_ref):
    z = h_ref[...] + (p0_ref[...] + p1_ref[...])
    z = jnp.dot(z, Wa_ref[...], preferred_element_type=jnp.float32) + ba_ref[...]
    z = jnp.maximum(z, 0.0)
    z = jnp.dot(z, Wb_ref[...], preferred_element_type=jnp.float32) + bb_ref[...]
    x3 = jnp.maximum(z, 0.0)
    ssum = x1_ref[...] + x2_ref[...] + x3
    p = jnp.sum(ssum * Wm_ref[...], axis=1, keepdims=True) + bm_ref[...]
    p = jnp.clip(p, -10.0, 10.0)
    o_ref[...] = 1.0 / (1.0 + jnp.exp(-p))


def _tc_head(h, p0, p1, x1, x2, Wa, ba, Wb, bb, Wm, bm):
    row = pl.BlockSpec((_ROW_BLK, D), lambda i: (i, 0))
    wsp = pl.BlockSpec((D, D), lambda i: (0, 0))
    bsp = pl.BlockSpec((1, D), lambda i: (0, 0))
    ssp = pl.BlockSpec((1, 1), lambda i: (0, 0))
    osp = pl.BlockSpec((_ROW_BLK, 1), lambda i: (i, 0))
    return pl.pallas_call(
        _head_body,
        grid=(N // _ROW_BLK,),
        in_specs=[row, row, row, row, row, wsp, bsp, wsp, bsp, bsp, ssp],
        out_specs=osp,
        out_shape=jax.ShapeDtypeStruct((N, 1), jnp.float32),
    )(h, p0, p1, x1, x2, Wa, ba.reshape(1, D), Wb, bb.reshape(1, D),
      Wm.reshape(1, D), bm.reshape(1, 1))


def kernel(x, edge_index, edge_attr, batch, imp_edge_index, graph_central_node,
           W0a, b0a, W0b, b0b, W1a, b1a, W1b, b1b, W2a, b2a, W2b, b2b, Wm, bm):
    pad = EPW_PAD - EPW
    src3d = jnp.pad(edge_index[0].reshape(NW, EPW),
                    ((0, 0), (0, pad))).reshape(NW, NCHUNK, CHUNK)
    dst3d = jnp.pad(edge_index[1].reshape(NW, EPW), ((0, 0), (0, pad)),
                    constant_values=N).reshape(NW, NCHUNK, CHUNK)
    zeros = jnp.zeros((NROW_ACC, D), jnp.float32)

    h = x
    parts = _sc_segment_sum(h, src3d, dst3d, zeros)
    x1 = _tc_mlp(h, parts[0, :N], parts[1, :N], W0a, b0a, W0b, b0b)
    parts = _sc_segment_sum(x1, src3d, dst3d, zeros)
    x2 = _tc_mlp(x1, parts[0, :N], parts[1, :N], W1a, b1a, W1b, b1b)
    parts = _sc_segment_sum(x2, src3d, dst3d, zeros)
    return _tc_head(x2, parts[0, :N], parts[1, :N], x1, x2,
                    W2a, b2a, W2b, b2b, Wm, bm)


# speedup vs baseline: 1.3418x; 1.3418x over previous
import functools

import jax
import jax.numpy as jnp
from jax import lax
from jax.experimental import pallas as pl
from jax.experimental.pallas import tpu as pltpu
from jax.experimental.pallas import tpu_sc as plsc

N = 10000
E = 320000
D = 128

NC = 2
NS = 16
NW = NC * NS
EPW = E // NW
CHUNK = 128
NCHUNK = -(-EPW // CHUNK)
EPW_PAD = NCHUNK * CHUNK
NROW_ACC = 10240
ROWS_PER_TILE = NROW_ACC // NS
OUT_CHUNK = 128


def _sc_segment_sum(h, src3d, dst3d, zeros):
    mesh = plsc.VectorSubcoreMesh(
        core_axis_name="c", subcore_axis_name="s", num_cores=NC,
        num_subcores=NS)

    @functools.partial(
        pl.kernel,
        out_type=jax.ShapeDtypeStruct((NC, NROW_ACC, D), jnp.float32),
        mesh=mesh,
        scratch_types=[
            pltpu.VMEM((NCHUNK, CHUNK), jnp.int32),
            pltpu.VMEM((NCHUNK, CHUNK), jnp.int32),
            pltpu.VMEM((CHUNK, D), jnp.float32),
            pltpu.VMEM_SHARED((NROW_ACC, D), jnp.float32),
            pltpu.SemaphoreType.DMA,
        ],
    )
    def seg_sum(h_hbm, src_hbm, dst_hbm, zeros_hbm, out_hbm,
                src_v, dst_v, rows_v, agg_sh, sem):
        c = lax.axis_index("c")
        s = lax.axis_index("s")
        w = c * NS + s
        base = s * ROWS_PER_TILE
        pltpu.sync_copy(zeros_hbm.at[pl.ds(base, ROWS_PER_TILE)],
                        agg_sh.at[pl.ds(base, ROWS_PER_TILE)])
        pltpu.sync_copy(src_hbm.at[w], src_v)
        pltpu.sync_copy(dst_hbm.at[w], dst_v)
        plsc.subcore_barrier()

        def body(j, carry):
            pltpu.async_copy(h_hbm.at[src_v.at[j]], rows_v, sem).wait()
            pltpu.sync_copy(rows_v, agg_sh.at[dst_v.at[j]], add=True)
            return carry

        lax.fori_loop(0, NCHUNK, body, 0)
        plsc.subcore_barrier()
        for k in range(ROWS_PER_TILE // OUT_CHUNK):
            off = base + k * OUT_CHUNK
            pltpu.sync_copy(agg_sh.at[pl.ds(off, OUT_CHUNK)], rows_v)
            pltpu.sync_copy(rows_v, out_hbm.at[c, pl.ds(off, OUT_CHUNK)])

    return seg_sum(h, src3d, dst3d, zeros)


_ROW_BLK = 1000


def _mlp_body(h_ref, p0_ref, p1_ref, Wa_ref, ba_ref, Wb_ref, bb_ref, o_ref):
    z = h_ref[...] + (p0_ref[...] + p1_ref[...])
    z = jnp.dot(z, Wa_ref[...], preferred_element_type=jnp.float32) + ba_ref[...]
    z = jnp.maximum(z, 0.0)
    z = jnp.dot(z, Wb_ref[...], preferred_element_type=jnp.float32) + bb_ref[...]
    o_ref[...] = jnp.maximum(z, 0.0)


def _tc_mlp(h, p0, p1, Wa, ba, Wb, bb):
    row = pl.BlockSpec((_ROW_BLK, D), lambda i: (i, 0))
    wsp = pl.BlockSpec((D, D), lambda i: (0, 0))
    bsp = pl.BlockSpec((1, D), lambda i: (0, 0))
    return pl.pallas_call(
        _mlp_body,
        grid=(N // _ROW_BLK,),
        in_specs=[row, row, row, wsp, bsp, wsp, bsp],
        out_specs=row,
        out_shape=jax.ShapeDtypeStruct((N, D), jnp.float32),
    )(h, p0, p1, Wa, ba.reshape(1, D), Wb, bb.reshape(1, D))


def _head_body(h_ref, p0_ref, p1_ref, x1_ref, x2_ref, Wa_ref, ba_ref,
               Wb_ref, bb_ref, Wm_ref, bm_ref, o_ref):
    z = h_ref[...] + (p0_ref[...] + p1_ref[...])
    z = jnp.dot(z, Wa_ref[...], preferred_element_type=jnp.float32) + ba_ref[...]
    z = jnp.maximum(z, 0.0)
    z = jnp.dot(z, Wb_ref[...], preferred_element_type=jnp.float32) + bb_ref[...]
    x3 = jnp.maximum(z, 0.0)
    ssum = x1_ref[...] + x2_ref[...] + x3
    p = jnp.sum(ssum * Wm_ref[...], axis=1, keepdims=True) + bm_ref[...]
    p = jnp.clip(p, -10.0, 10.0)
    o_ref[...] = 1.0 / (1.0 + jnp.exp(-p))


def _tc_head(h, p0, p1, x1, x2, Wa, ba, Wb, bb, Wm, bm):
    row = pl.BlockSpec((_ROW_BLK, D), lambda i: (i, 0))
    wsp = pl.BlockSpec((D, D), lambda i: (0, 0))
    bsp = pl.BlockSpec((1, D), lambda i: (0, 0))
    ssp = pl.BlockSpec((1, 1), lambda i: (0, 0))
    osp = pl.BlockSpec((_ROW_BLK, 1), lambda i: (i, 0))
    return pl.pallas_call(
        _head_body,
        grid=(N // _ROW_BLK,),
        in_specs=[row, row, row, row, row, wsp, bsp, wsp, bsp, bsp, ssp],
        out_specs=osp,
        out_shape=jax.ShapeDtypeStruct((N, 1), jnp.float32),
    )(h, p0, p1, x1, x2, Wa, ba.reshape(1, D), Wb, bb.reshape(1, D),
      Wm.reshape(1, D), bm.reshape(1, 1))


def kernel(x, edge_index, edge_attr, batch, imp_edge_index, graph_central_node,
           W0a, b0a, W0b, b0b, W1a, b1a, W1b, b1b, W2a, b2a, W2b, b2b, Wm, bm):
    pad = EPW_PAD - EPW
    src3d = jnp.pad(edge_index[0].reshape(NW, EPW),
                    ((0, 0), (0, pad))).reshape(NW, NCHUNK, CHUNK)
    dst3d = jnp.pad(edge_index[1].reshape(NW, EPW), ((0, 0), (0, pad)),
                    constant_values=N).reshape(NW, NCHUNK, CHUNK)
    zeros = jnp.zeros((NROW_ACC, D), jnp.float32)

    h = x
    parts = _sc_segment_sum(h, src3d, dst3d, zeros)
    x1 = _tc_mlp(h, parts[0, :N], parts[1, :N], W0a, b0a, W0b, b0b)
    parts = _sc_segment_sum(x1, src3d, dst3d, zeros)
    x2 = _tc_mlp(x1, parts[0, :N], parts[1, :N], W1a, b1a, W1b, b1b)
    parts = _sc_segment_sum(x2, src3d, dst3d, zeros)
    return _tc_head(x2, parts[0, :N], parts[1, :N], x1, x2,
                    W2a, b2a, W2b, b2b, Wm, bm)
